# baseline (speedup 1.0000x reference)
import functools

import jax
import jax.numpy as jnp
import numpy as np
from jax import lax
from jax.experimental import pallas as pl
from jax.experimental.pallas import tpu as pltpu
from jax.experimental.pallas import tpu_sc as plsc

_RANK = 100000

_NC, _NS, _L = 2, 16, 16
_NW = _NC * _NS
_N_PER = 4 * 512 * 512
_CHUNK = _N_PER // _NW
_NBINS = 65536


def _keys_of(b_i32):
    return b_i32 ^ ((b_i32 >> np.int32(31)) & np.int32(0x7FFFFFFF))


def _sc_hist_body(e_hbm, m_hbm, d_hbm, meta_hbm, out_hbm, buf, hist, hvec,
                  *, low_pass):
    wid = lax.axis_index("s") * _NC + lax.axis_index("c")
    base = wid * _CHUNK

    @pl.loop(0, _NBINS // _L)
    def _zero(i):
        hist[pl.ds(i * _L, _L)] = jnp.zeros((_L,), jnp.int32)

    if low_pass:
        pltpu.sync_copy(meta_hbm.at[pl.ds(0, _L)], hvec)
        hstar = hvec[...]
    ones = jnp.ones((_L,), jnp.int32)

    for src in (e_hbm, m_hbm, d_hbm):
        pltpu.sync_copy(src.at[pl.ds(base, _CHUNK)], buf)

        @pl.loop(0, _CHUNK // _L)
        def _accum(i):
            key = _keys_of(buf[pl.ds(i * _L, _L)])
            high = (key >> np.int32(16)) + np.int32(32768)
            if low_pass:
                low = key & np.int32(0xFFFF)
                plsc.addupdate_scatter(hist, [low], ones, mask=high == hstar)
            else:
                plsc.addupdate_scatter(hist, [high], ones)

    pltpu.sync_copy(hist, out_hbm.at[wid])


def _make_sc_hist(low_pass):
    mesh = plsc.VectorSubcoreMesh(core_axis_name="c", subcore_axis_name="s",
                                  num_cores=_NC, num_subcores=_NS)
    return pl.kernel(
        functools.partial(_sc_hist_body, low_pass=low_pass),
        out_type=jax.ShapeDtypeStruct((_NW, _NBINS), jnp.int32),
        mesh=mesh,
        compiler_params=pltpu.CompilerParams(needs_layout_passes=False),
        scratch_types=[
            pltpu.VMEM((_CHUNK,), jnp.int32),
            pltpu.VMEM((_NBINS,), jnp.int32),
            pltpu.VMEM((_L,), jnp.int32),
        ],
    )


def _search_hist(hists_i32, k_f32):
    h = jnp.sum(hists_i32.astype(jnp.float32), axis=0)
    tri512 = (lax.broadcasted_iota(jnp.int32, (512, 512), 0)
              > lax.broadcasted_iota(jnp.int32, (512, 512), 1)
              ).astype(jnp.float32)
    tri128 = (lax.broadcasted_iota(jnp.int32, (128, 128), 0)
              > lax.broadcasted_iota(jnp.int32, (128, 128), 1)
              ).astype(jnp.float32)

    dot = functools.partial(jnp.dot, precision=lax.Precision.HIGHEST,
                            preferred_element_type=jnp.float32)
    rsum = jnp.sum(h, axis=1)[None, :]
    srow = dot(rsum, tri512)
    row_hit = ((srow <= k_f32) & (srow + rsum > k_f32)).astype(jnp.float32)
    iota_row = lax.broadcasted_iota(jnp.int32, (1, 512), 1).astype(jnp.float32)
    rstar = jnp.sum(row_hit * iota_row)
    sbase = jnp.sum(row_hit * srow)

    colvec = dot(row_hit, h)
    scol = dot(colvec, tri128)
    tot = sbase + scol
    col_hit = ((tot <= k_f32) & (tot + colvec > k_f32)).astype(jnp.float32)
    iota_col = lax.broadcasted_iota(jnp.int32, (1, 128), 1).astype(jnp.float32)
    cstar = jnp.sum(col_hit * iota_col)
    sstar = sbase + jnp.sum(col_hit * scol)
    return rstar * 128.0 + cstar, sstar


def _tc_find_bin_body(hists_ref, meta_ref):
    hstar, sstar = _search_hist(hists_ref[...], jnp.float32(_RANK))
    kprime = jnp.float32(_RANK) - sstar
    row = lax.broadcasted_iota(jnp.int32, (8, 128), 0)
    meta_ref[...] = jnp.where(row == 0, hstar.astype(jnp.int32),
                              kprime.astype(jnp.int32))


def _tc_thresh_body(hists_ref, meta_ref, t_ref):
    kprime = meta_ref[1, 0].astype(jnp.float32)
    hstar = meta_ref[0, 0]
    lowstar, _ = _search_hist(hists_ref[...], kprime)
    t = ((hstar - np.int32(32768)) << np.int32(16)) | lowstar.astype(jnp.int32)
    bits = t ^ ((t >> np.int32(31)) & np.int32(0x7FFFFFFF))
    thresh = lax.bitcast_convert_type(bits, jnp.float32)
    t_ref[...] = jnp.full((8, 128), 1.0, jnp.float32) * thresh


def _tc_mask_body(t_ref, e_ref, m_ref, d_ref, oe_ref, om_ref, od_ref):
    t = t_ref[0, 0]
    for src, dst in ((e_ref, oe_ref), (m_ref, om_ref), (d_ref, od_ref)):
        x = src[...]
        dst[...] = jnp.where(x < t, jnp.float32(0.0), x)


def kernel(early, middle, deep):
    shp = early.shape
    eb = lax.bitcast_convert_type(early, jnp.int32).reshape(-1)
    mb = lax.bitcast_convert_type(middle, jnp.int32).reshape(-1)
    db = lax.bitcast_convert_type(deep, jnp.int32).reshape(-1)
    unused_meta = jnp.zeros((1024,), jnp.int32)

    hist1 = _make_sc_hist(low_pass=False)(eb, mb, db, unused_meta)

    meta = pl.pallas_call(
        _tc_find_bin_body,
        out_shape=jax.ShapeDtypeStruct((8, 128), jnp.int32),
    )(hist1.reshape(_NW, 512, 128))

    hist2 = _make_sc_hist(low_pass=True)(eb, mb, db, meta.reshape(-1))

    thresh = pl.pallas_call(
        _tc_thresh_body,
        out_shape=jax.ShapeDtypeStruct((8, 128), jnp.float32),
    )(hist2.reshape(_NW, 512, 128), meta)

    oe, om, od = pl.pallas_call(
        _tc_mask_body,
        out_shape=tuple(
            jax.ShapeDtypeStruct((1024, 1024), jnp.float32) for _ in range(3)),
    )(thresh, early.reshape(1024, 1024), middle.reshape(1024, 1024),
      deep.reshape(1024, 1024))

    return (oe.reshape(shp), om.reshape(shp), od.reshape(shp))


# speedup vs baseline: 1.3620x; 1.3620x over previous
import functools

import jax
import jax.numpy as jnp
import numpy as np
from jax import lax
from jax.experimental import pallas as pl
from jax.experimental.pallas import tpu as pltpu
from jax.experimental.pallas import tpu_sc as plsc

_RANK = 100000

_NC, _NS, _L = 2, 16, 16
_NW = _NC * _NS
_N_PER = 4 * 512 * 512
_CHUNK = _N_PER // _NW
_HALF = _CHUNK // 2
_NBINS = 65536


def _keys_of(b_i32):
    return b_i32 ^ ((b_i32 >> np.int32(31)) & np.int32(0x7FFFFFFF))


def _sc_hist_common(e_hbm, m_hbm, d_hbm, meta_hbm, out_hbm,
                    buf0, buf1, hist, hvec, idx, shist, sem0, sem1,
                    *, low_pass):
    cid = lax.axis_index("c")
    sid = lax.axis_index("s")
    wid = sid * _NC + cid
    base = wid * _CHUNK

    @pl.loop(0, 512, unroll=8)
    def _zero(r):
        for c in range(8):
            hist[r, pl.ds(c * _L, _L)] = jnp.zeros((_L,), jnp.int32)

    for j in range(4):
        for c in range(8):
            idx[j, pl.ds(c * _L, _L)] = (
                lax.broadcasted_iota(jnp.int32, (_L,), 0)
                + np.int32(j * 128 + c * _L))

    @pl.when(sid == 0)
    def _():
        pltpu.sync_copy(hist, shist)

    if low_pass:
        pltpu.sync_copy(meta_hbm.at[pl.ds(0, _L)], hvec)
        hs = hvec[...] - np.int32(32768)
    ones = jnp.ones((_L,), jnp.int32)

    bufs = (buf0, buf1)
    sems = (sem0, sem1)
    srcs = []
    for arr in (e_hbm, m_hbm, d_hbm):
        for h in range(2):
            srcs.append(arr.at[pl.ds(base + h * _HALF, _HALF)])

    copies = [None, None]
    copies[0] = pltpu.async_copy(srcs[0], bufs[0], sems[0])
    for q in range(len(srcs)):
        copies[q % 2].wait()
        if q + 1 < len(srcs):
            copies[(q + 1) % 2] = pltpu.async_copy(
                srcs[q + 1], bufs[(q + 1) % 2], sems[(q + 1) % 2])
        buf = bufs[q % 2]

        @pl.loop(0, _HALF // _L, unroll=8)
        def _accum(i):
            key = _keys_of(buf[pl.ds(i * _L, _L)])
            if low_pass:
                row = (key >> np.int32(7)) & np.int32(511)
                col = key & np.int32(127)
                plsc.addupdate_scatter(
                    hist, [row, col], ones,
                    mask=(key >> np.int32(16)) == hs)
            else:
                row = (key >> np.int32(23)) + np.int32(256)
                col = (key >> np.int32(16)) & np.int32(127)
                plsc.addupdate_scatter(hist, [row, col], ones)

    plsc.subcore_barrier()
    for j in range(4):
        pltpu.sync_copy(hist.at[pl.ds(j * 128, 128)],
                        shist.at[idx.at[j]], add=True)
    plsc.subcore_barrier()

    @pl.when(sid == 0)
    def _():
        pltpu.sync_copy(shist, out_hbm.at[cid])


def _make_sc_hist(low_pass):
    mesh = plsc.VectorSubcoreMesh(core_axis_name="c", subcore_axis_name="s",
                                  num_cores=_NC, num_subcores=_NS)
    return pl.kernel(
        functools.partial(_sc_hist_common, low_pass=low_pass),
        out_type=jax.ShapeDtypeStruct((_NC, 512, 128), jnp.int32),
        mesh=mesh,
        compiler_params=pltpu.CompilerParams(needs_layout_passes=False),
        scratch_types=[
            pltpu.VMEM((_HALF,), jnp.int32),
            pltpu.VMEM((_HALF,), jnp.int32),
            pltpu.VMEM((512, 128), jnp.int32),
            pltpu.VMEM((_L,), jnp.int32),
            pltpu.VMEM((4, 128), jnp.int32),
            pltpu.VMEM_SHARED((512, 128), jnp.int32),
            pltpu.SemaphoreType.DMA,
            pltpu.SemaphoreType.DMA,
        ],
    )


def _search_hist(hists_i32, k_f32):
    h = jnp.sum(hists_i32.astype(jnp.float32), axis=0)
    tri512 = (lax.broadcasted_iota(jnp.int32, (512, 512), 0)
              > lax.broadcasted_iota(jnp.int32, (512, 512), 1)
              ).astype(jnp.float32)
    tri128 = (lax.broadcasted_iota(jnp.int32, (128, 128), 0)
              > lax.broadcasted_iota(jnp.int32, (128, 128), 1)
              ).astype(jnp.float32)

    dot = functools.partial(jnp.dot, precision=lax.Precision.HIGHEST,
                            preferred_element_type=jnp.float32)
    rsum = jnp.sum(h, axis=1)[None, :]
    srow = dot(rsum, tri512)
    row_hit = ((srow <= k_f32) & (srow + rsum > k_f32)).astype(jnp.float32)
    iota_row = lax.broadcasted_iota(jnp.int32, (1, 512), 1).astype(jnp.float32)
    rstar = jnp.sum(row_hit * iota_row)
    sbase = jnp.sum(row_hit * srow)

    colvec = dot(row_hit, h)
    scol = dot(colvec, tri128)
    tot = sbase + scol
    col_hit = ((tot <= k_f32) & (tot + colvec > k_f32)).astype(jnp.float32)
    iota_col = lax.broadcasted_iota(jnp.int32, (1, 128), 1).astype(jnp.float32)
    cstar = jnp.sum(col_hit * iota_col)
    sstar = sbase + jnp.sum(col_hit * scol)
    return rstar * 128.0 + cstar, sstar


def _tc_find_bin_body(hists_ref, meta_ref):
    hstar, sstar = _search_hist(hists_ref[...], jnp.float32(_RANK))
    kprime = jnp.float32(_RANK) - sstar
    row = lax.broadcasted_iota(jnp.int32, (8, 128), 0)
    meta_ref[...] = jnp.where(row == 0, hstar.astype(jnp.int32),
                              kprime.astype(jnp.int32))


def _tc_thresh_mask_body(hists_ref, meta_ref, e_ref, m_ref, d_ref,
                         oe_ref, om_ref, od_ref):
    kprime = meta_ref[1, 0].astype(jnp.float32)
    hstar = meta_ref[0, 0]
    lowstar, _ = _search_hist(hists_ref[...], kprime)
    t = ((hstar - np.int32(32768)) << np.int32(16)) | lowstar.astype(jnp.int32)
    bits = t ^ ((t >> np.int32(31)) & np.int32(0x7FFFFFFF))
    thresh = lax.bitcast_convert_type(bits, jnp.float32)
    for src, dst in ((e_ref, oe_ref), (m_ref, om_ref), (d_ref, od_ref)):
        x = src[...]
        dst[...] = jnp.where(x < thresh, jnp.float32(0.0), x)


def kernel(early, middle, deep):
    shp = early.shape
    eb = lax.bitcast_convert_type(early, jnp.int32).reshape(-1)
    mb = lax.bitcast_convert_type(middle, jnp.int32).reshape(-1)
    db = lax.bitcast_convert_type(deep, jnp.int32).reshape(-1)
    unused_meta = jnp.zeros((1024,), jnp.int32)

    hist1 = _make_sc_hist(low_pass=False)(eb, mb, db, unused_meta)

    meta = pl.pallas_call(
        _tc_find_bin_body,
        out_shape=jax.ShapeDtypeStruct((8, 128), jnp.int32),
    )(hist1)

    hist2 = _make_sc_hist(low_pass=True)(eb, mb, db, meta.reshape(-1))

    oe, om, od = pl.pallas_call(
        _tc_thresh_mask_body,
        out_shape=tuple(
            jax.ShapeDtypeStruct((1024, 1024), jnp.float32) for _ in range(3)),
    )(hist2, meta, early.reshape(1024, 1024), middle.reshape(1024, 1024),
      deep.reshape(1024, 1024))

    return (oe.reshape(shp), om.reshape(shp), od.reshape(shp))


# speedup vs baseline: 2.7561x; 2.0236x over previous
import functools

import jax
import jax.numpy as jnp
import numpy as np
from jax import lax
from jax.experimental import pallas as pl
from jax.experimental.pallas import tpu as pltpu
from jax.experimental.pallas import tpu_sc as plsc

_RANK = 100000

_NC, _NS, _L = 2, 16, 16
_NW = _NC * _NS
_N_PER = 4 * 512 * 512
_CHUNK = _N_PER // _NW
_HALF = _CHUNK // 2
_NBINS = 65536


def _keys_of(b_i32):
    return b_i32 ^ ((b_i32 >> np.int32(31)) & np.int32(0x7FFFFFFF))


def _sc_hist_common(e_hbm, m_hbm, d_hbm, meta_hbm, out_hbm,
                    buf0, buf1, hist, hvec, idx, shist, sem0, sem1,
                    *, low_pass):
    cid = lax.axis_index("c")
    sid = lax.axis_index("s")
    wid = sid * _NC + cid
    base = wid * _CHUNK

    @plsc.parallel_loop(0, 512, unroll=8)
    def _zero(r):
        for c in range(8):
            hist[r, pl.ds(c * _L, _L)] = jnp.zeros((_L,), jnp.int32)

    for j in range(4):
        for c in range(8):
            idx[j, pl.ds(c * _L, _L)] = (
                lax.broadcasted_iota(jnp.int32, (_L,), 0)
                + np.int32(j * 128 + c * _L))

    @pl.when(sid == 0)
    def _():
        pltpu.sync_copy(hist, shist)

    if low_pass:
        pltpu.sync_copy(meta_hbm.at[pl.ds(0, _L)], hvec)
        hs = hvec[...] - np.int32(32768)
    ones = jnp.ones((_L,), jnp.int32)

    bufs = (buf0, buf1)
    sems = (sem0, sem1)
    srcs = []
    for arr in (e_hbm, m_hbm, d_hbm):
        for h in range(2):
            srcs.append(arr.at[pl.ds(base + h * _HALF, _HALF)])

    copies = [None, None]
    copies[0] = pltpu.async_copy(srcs[0], bufs[0], sems[0])
    for q in range(len(srcs)):
        copies[q % 2].wait()
        if q + 1 < len(srcs):
            copies[(q + 1) % 2] = pltpu.async_copy(
                srcs[q + 1], bufs[(q + 1) % 2], sems[(q + 1) % 2])
        buf = bufs[q % 2]

        @plsc.parallel_loop(0, _HALF // _L, unroll=8)
        def _accum(i):
            key = _keys_of(buf[pl.ds(i * _L, _L)])
            if low_pass:
                row = (key >> np.int32(7)) & np.int32(511)
                col = key & np.int32(127)
                plsc.addupdate_scatter(
                    hist, [row, col], ones,
                    mask=(key >> np.int32(16)) == hs)
            else:
                row = (key >> np.int32(23)) + np.int32(256)
                col = (key >> np.int32(16)) & np.int32(127)
                plsc.addupdate_scatter(hist, [row, col], ones)

    plsc.subcore_barrier()
    for j in range(4):
        pltpu.sync_copy(hist.at[pl.ds(j * 128, 128)],
                        shist.at[idx.at[j]], add=True)
    plsc.subcore_barrier()

    @pl.when(sid == 0)
    def _():
        pltpu.sync_copy(shist, out_hbm.at[cid])


def _make_sc_hist(low_pass):
    mesh = plsc.VectorSubcoreMesh(core_axis_name="c", subcore_axis_name="s",
                                  num_cores=_NC, num_subcores=_NS)
    return pl.kernel(
        functools.partial(_sc_hist_common, low_pass=low_pass),
        out_type=jax.ShapeDtypeStruct((_NC, 512, 128), jnp.int32),
        mesh=mesh,
        compiler_params=pltpu.CompilerParams(needs_layout_passes=False),
        scratch_types=[
            pltpu.VMEM((_HALF,), jnp.int32),
            pltpu.VMEM((_HALF,), jnp.int32),
            pltpu.VMEM((512, 128), jnp.int32),
            pltpu.VMEM((_L,), jnp.int32),
            pltpu.VMEM((4, 128), jnp.int32),
            pltpu.VMEM_SHARED((512, 128), jnp.int32),
            pltpu.SemaphoreType.DMA,
            pltpu.SemaphoreType.DMA,
        ],
    )


def _search_hist(hists_i32, k_f32):
    h = jnp.sum(hists_i32.astype(jnp.float32), axis=0)
    tri512 = (lax.broadcasted_iota(jnp.int32, (512, 512), 0)
              > lax.broadcasted_iota(jnp.int32, (512, 512), 1)
              ).astype(jnp.float32)
    tri128 = (lax.broadcasted_iota(jnp.int32, (128, 128), 0)
              > lax.broadcasted_iota(jnp.int32, (128, 128), 1)
              ).astype(jnp.float32)

    dot = functools.partial(jnp.dot, precision=lax.Precision.HIGHEST,
                            preferred_element_type=jnp.float32)
    rsum = jnp.sum(h, axis=1)[None, :]
    srow = dot(rsum, tri512)
    row_hit = ((srow <= k_f32) & (srow + rsum > k_f32)).astype(jnp.float32)
    iota_row = lax.broadcasted_iota(jnp.int32, (1, 512), 1).astype(jnp.float32)
    rstar = jnp.sum(row_hit * iota_row)
    sbase = jnp.sum(row_hit * srow)

    colvec = dot(row_hit, h)
    scol = dot(colvec, tri128)
    tot = sbase + scol
    col_hit = ((tot <= k_f32) & (tot + colvec > k_f32)).astype(jnp.float32)
    iota_col = lax.broadcasted_iota(jnp.int32, (1, 128), 1).astype(jnp.float32)
    cstar = jnp.sum(col_hit * iota_col)
    sstar = sbase + jnp.sum(col_hit * scol)
    return rstar * 128.0 + cstar, sstar


def _tc_find_bin_body(hists_ref, meta_ref):
    hstar, sstar = _search_hist(hists_ref[...], jnp.float32(_RANK))
    kprime = jnp.float32(_RANK) - sstar
    row = lax.broadcasted_iota(jnp.int32, (8, 128), 0)
    meta_ref[...] = jnp.where(row == 0, hstar.astype(jnp.int32),
                              kprime.astype(jnp.int32))


def _tc_thresh_mask_body(hists_ref, meta_ref, e_ref, m_ref, d_ref,
                         oe_ref, om_ref, od_ref):
    kprime = meta_ref[1, 0].astype(jnp.float32)
    hstar = meta_ref[0, 0]
    lowstar, _ = _search_hist(hists_ref[...], kprime)
    t = ((hstar - np.int32(32768)) << np.int32(16)) | lowstar.astype(jnp.int32)
    bits = t ^ ((t >> np.int32(31)) & np.int32(0x7FFFFFFF))
    thresh = lax.bitcast_convert_type(bits, jnp.float32)
    for src, dst in ((e_ref, oe_ref), (m_ref, om_ref), (d_ref, od_ref)):
        x = src[...]
        dst[...] = jnp.where(x < thresh, jnp.float32(0.0), x)


def kernel(early, middle, deep):
    shp = early.shape
    eb = lax.bitcast_convert_type(early, jnp.int32).reshape(-1)
    mb = lax.bitcast_convert_type(middle, jnp.int32).reshape(-1)
    db = lax.bitcast_convert_type(deep, jnp.int32).reshape(-1)
    unused_meta = jnp.zeros((1024,), jnp.int32)

    hist1 = _make_sc_hist(low_pass=False)(eb, mb, db, unused_meta)

    meta = pl.pallas_call(
        _tc_find_bin_body,
        out_shape=jax.ShapeDtypeStruct((8, 128), jnp.int32),
    )(hist1)

    hist2 = _make_sc_hist(low_pass=True)(eb, mb, db, meta.reshape(-1))

    oe, om, od = pl.pallas_call(
        _tc_thresh_mask_body,
        out_shape=tuple(
            jax.ShapeDtypeStruct((1024, 1024), jnp.float32) for _ in range(3)),
    )(hist2, meta, early.reshape(1024, 1024), middle.reshape(1024, 1024),
      deep.reshape(1024, 1024))

    return (oe.reshape(shp), om.reshape(shp), od.reshape(shp))
